# baseline (speedup 1.0000x reference)
import jax
import jax.numpy as jnp
from jax import lax
from jax.experimental import pallas as pl
from jax.experimental.pallas import tpu as pltpu
from jax.experimental.pallas import tpu_sc as plsc

NUM_MOTIFS = 1000000
EMBED_DIM = 64
BIO_DIM = 128
B, L = 4096, 50
TOTAL = B * L

NC = 2
NS = 16
NW = NC * NS
PER_W = TOTAL // NW
CHUNK = 128
ITERS = PER_W // CHUNK

T_BLK = 8192
T_GRID = -(-NUM_MOTIFS // T_BLK)


def _transpose_body(tt_ref, eye_ref, out_ref):
    xt = jax.lax.dot_general(
        tt_ref[...], eye_ref[...], (((0,), (0,)), ((), ())),
        preferred_element_type=jnp.float32,
    )
    out_ref[...] = jnp.concatenate(
        [xt[: T_BLK // 2], xt[T_BLK // 2:]], axis=1
    )


def _tc_transpose(table_t, eye):
    return pl.pallas_call(
        _transpose_body,
        grid=(T_GRID,),
        in_specs=[
            pl.BlockSpec((EMBED_DIM, T_BLK), lambda i: (0, i)),
            pl.BlockSpec((EMBED_DIM, EMBED_DIM), lambda i: (0, 0)),
        ],
        out_specs=pl.BlockSpec((T_BLK // 2, 2 * EMBED_DIM), lambda i: (i, 0)),
        out_shape=jax.ShapeDtypeStruct(
            (T_GRID * (T_BLK // 2), 2 * EMBED_DIM), jnp.float32
        ),
    )(table_t, eye)


def _sc_gather_body(table_hbm, ids_hbm, out_hbm, idx_v, buf0, buf1,
                    gsem0, gsem1, ssem0, ssem1):
    wid = lax.axis_index("s") * NC + lax.axis_index("c")
    base = wid * PER_W
    pltpu.sync_copy(ids_hbm.at[wid], idx_v)

    bufs = (buf0, buf1)
    gsems = (gsem0, gsem1)
    ssems = (ssem0, ssem1)

    def store(g):
        return pltpu.async_copy(
            bufs[g % 2],
            out_hbm.at[pl.ds(base + g * CHUNK, CHUNK)],
            ssems[g % 2],
        )

    def gather(g):
        return pltpu.async_copy(table_hbm.at[idx_v.at[g]], bufs[g % 2],
                                gsems[g % 2])

    hg = [None] * ITERS
    hs = [None] * ITERS
    for g in range(ITERS):
        if g >= 2:
            hs[g - 2].wait()
        hg[g] = gather(g)
        if g >= 1:
            hg[g - 1].wait()
            hs[g - 1] = store(g - 1)
    hg[ITERS - 1].wait()
    hs[ITERS - 1] = store(ITERS - 1)
    hs[ITERS - 2].wait()
    hs[ITERS - 1].wait()


def _sc_gather(table2, ids3d):
    mesh = plsc.VectorSubcoreMesh(
        core_axis_name="c", subcore_axis_name="s", num_cores=NC, num_subcores=NS
    )
    return pl.kernel(
        _sc_gather_body,
        out_type=jax.ShapeDtypeStruct((TOTAL, 2 * EMBED_DIM), jnp.float32),
        mesh=mesh,
        scratch_types=[
            pltpu.VMEM((ITERS, CHUNK), jnp.int32),
            pltpu.VMEM((CHUNK, 2 * EMBED_DIM), jnp.float32),
            pltpu.VMEM((CHUNK, 2 * EMBED_DIM), jnp.float32),
            pltpu.SemaphoreType.DMA,
            pltpu.SemaphoreType.DMA,
            pltpu.SemaphoreType.DMA,
            pltpu.SemaphoreType.DMA,
        ],
        compiler_params=pltpu.CompilerParams(use_tc_tiling_on_sc=False),
    )(table2, ids3d)


def _proj_body(bio_ref, wt_ref, b_ref, out_ref):
    acc = jnp.dot(bio_ref[...], wt_ref[...], preferred_element_type=jnp.float32)
    out_ref[...] = (acc + b_ref[...]).T[None]


def _tc_proj(bio2d, wt, b2d):
    return pl.pallas_call(
        _proj_body,
        grid=(L,),
        in_specs=[
            pl.BlockSpec((B, BIO_DIM), lambda i: (i, 0)),
            pl.BlockSpec((BIO_DIM, EMBED_DIM), lambda i: (0, 0)),
            pl.BlockSpec((1, EMBED_DIM), lambda i: (0, 0)),
        ],
        out_specs=pl.BlockSpec((1, EMBED_DIM, B), lambda i: (i, 0, 0)),
        out_shape=jax.ShapeDtypeStruct((L, EMBED_DIM, B), jnp.float32),
    )(bio2d, wt, b2d)


def _add_body(proj_ref, g_ref, par_ref, out_ref):
    odd_col = (par_ref[0] == 1).T
    emb = jnp.where(odd_col, g_ref[:, EMBED_DIM:], g_ref[:, :EMBED_DIM])
    out_ref[...] = (proj_ref[0] + emb.T)[None]


def _tc_add(proj_t, gathered, parity):
    return pl.pallas_call(
        _add_body,
        grid=(L,),
        in_specs=[
            pl.BlockSpec((1, EMBED_DIM, B), lambda i: (i, 0, 0)),
            pl.BlockSpec((B, 2 * EMBED_DIM), lambda i: (i, 0)),
            pl.BlockSpec((1, 1, B), lambda i: (i, 0, 0)),
        ],
        out_specs=pl.BlockSpec((1, EMBED_DIM, B), lambda i: (i, 0, 0)),
        out_shape=jax.ShapeDtypeStruct((L, EMBED_DIM, B), jnp.float32),
    )(proj_t, gathered, parity)


@jax.jit
def kernel(motif_ids, biological_features, emb_table, W, b):
    table2 = _tc_transpose(emb_table.T, jnp.eye(EMBED_DIM, dtype=jnp.float32))
    ids_t = motif_ids.T.astype(jnp.int32)
    ids3d = (((ids_t >> 13) << 12) | (ids_t & 4095)).reshape(NW, ITERS, CHUNK)
    parity = ((ids_t >> 12) & 1).reshape(L, 1, B)
    gathered = _sc_gather(table2, ids3d)
    bio2d = biological_features.transpose(1, 0, 2).reshape(TOTAL, BIO_DIM)
    proj_t = _tc_proj(bio2d, W.T, b.reshape(1, EMBED_DIM))
    out_t = _tc_add(proj_t, gathered, parity)
    return out_t.transpose(2, 0, 1)


# speedup vs baseline: 1.2684x; 1.2684x over previous
import jax
import jax.numpy as jnp
from jax import lax
from jax.experimental import pallas as pl
from jax.experimental.pallas import tpu as pltpu
from jax.experimental.pallas import tpu_sc as plsc

NUM_MOTIFS = 1000000
EMBED_DIM = 64
BIO_DIM = 128
B, L = 4096, 50
TOTAL = B * L

NC = 2
NS = 16
NW = NC * NS
PER_W = TOTAL // NW
CHUNK = 128
ITERS = PER_W // CHUNK

T_BLK = 16384
T_GRID = -(-NUM_MOTIFS // T_BLK)


def _transpose_body(tt_ref, out_ref):
    xt = tt_ref[...].T
    out_ref[...] = jnp.concatenate(
        [xt, jnp.zeros((T_BLK, 2 * EMBED_DIM - EMBED_DIM), jnp.float32)], axis=1
    )


def _tc_transpose(table_t):
    return pl.pallas_call(
        _transpose_body,
        grid=(T_GRID,),
        in_specs=[pl.BlockSpec((EMBED_DIM, T_BLK), lambda i: (0, i))],
        out_specs=pl.BlockSpec((T_BLK, 2 * EMBED_DIM), lambda i: (i, 0)),
        out_shape=jax.ShapeDtypeStruct((NUM_MOTIFS, 2 * EMBED_DIM), jnp.float32),
    )(table_t)


def _sc_gather_body(table_hbm, ids_hbm, out_hbm, idx_v, buf0, buf1,
                    gsem0, gsem1, ssem0, ssem1):
    wid = lax.axis_index("s") * NC + lax.axis_index("c")
    base = wid * PER_W
    pltpu.sync_copy(ids_hbm.at[wid], idx_v)

    bufs = (buf0, buf1)
    gsems = (gsem0, gsem1)
    ssems = (ssem0, ssem1)

    def store(g):
        return pltpu.async_copy(
            bufs[g % 2].at[:, pl.ds(0, EMBED_DIM)],
            out_hbm.at[pl.ds(base + g * CHUNK, CHUNK), pl.ds(0, EMBED_DIM)],
            ssems[g % 2],
        )

    def gather(g):
        return pltpu.async_copy(table_hbm.at[idx_v.at[g]], bufs[g % 2],
                                gsems[g % 2])

    hg = [None] * ITERS
    hs = [None] * ITERS
    for g in range(ITERS):
        if g >= 2:
            hs[g - 2].wait()
        hg[g] = gather(g)
        if g >= 1:
            hg[g - 1].wait()
            hs[g - 1] = store(g - 1)
    hg[ITERS - 1].wait()
    hs[ITERS - 1] = store(ITERS - 1)
    hs[ITERS - 2].wait()
    hs[ITERS - 1].wait()


def _sc_gather(table128, ids3d):
    mesh = plsc.VectorSubcoreMesh(
        core_axis_name="c", subcore_axis_name="s", num_cores=NC, num_subcores=NS
    )
    return pl.kernel(
        _sc_gather_body,
        out_type=jax.ShapeDtypeStruct((TOTAL, 2 * EMBED_DIM), jnp.float32),
        mesh=mesh,
        scratch_types=[
            pltpu.VMEM((ITERS, CHUNK), jnp.int32),
            pltpu.VMEM((CHUNK, 2 * EMBED_DIM), jnp.float32),
            pltpu.VMEM((CHUNK, 2 * EMBED_DIM), jnp.float32),
            pltpu.SemaphoreType.DMA,
            pltpu.SemaphoreType.DMA,
            pltpu.SemaphoreType.DMA,
            pltpu.SemaphoreType.DMA,
        ],
        compiler_params=pltpu.CompilerParams(use_tc_tiling_on_sc=False),
    )(table128, ids3d)


def _tc_body(bio_ref, wt_ref, b_ref, g_ref, out_ref):
    acc = jnp.dot(bio_ref[...], wt_ref[...], preferred_element_type=jnp.float32)
    acc = acc + b_ref[...] + g_ref[:, :EMBED_DIM]
    out_ref[...] = acc.T[None]


def _tc_proj_add(bio2d, wt, b2d, gathered):
    return pl.pallas_call(
        _tc_body,
        grid=(L,),
        in_specs=[
            pl.BlockSpec((B, BIO_DIM), lambda i: (i, 0)),
            pl.BlockSpec((BIO_DIM, EMBED_DIM), lambda i: (0, 0)),
            pl.BlockSpec((1, EMBED_DIM), lambda i: (0, 0)),
            pl.BlockSpec((B, 2 * EMBED_DIM), lambda i: (i, 0)),
        ],
        out_specs=pl.BlockSpec((1, EMBED_DIM, B), lambda i: (i, 0, 0)),
        out_shape=jax.ShapeDtypeStruct((L, EMBED_DIM, B), jnp.float32),
    )(bio2d, wt, b2d, gathered)


@jax.jit
def kernel(motif_ids, biological_features, emb_table, W, b):
    table128 = _tc_transpose(emb_table.T)
    ids3d = motif_ids.T.reshape(NW, ITERS, CHUNK).astype(jnp.int32)
    gathered = _sc_gather(table128, ids3d)
    bio2d = biological_features.transpose(1, 0, 2).reshape(TOTAL, BIO_DIM)
    out_t = _tc_proj_add(bio2d, W.T, b.reshape(1, EMBED_DIM), gathered)
    return out_t.transpose(2, 0, 1)


# speedup vs baseline: 1.2854x; 1.0134x over previous
import jax
import jax.numpy as jnp
from jax import lax
from jax.experimental import pallas as pl
from jax.experimental.pallas import tpu as pltpu
from jax.experimental.pallas import tpu_sc as plsc

NUM_MOTIFS = 1000000
EMBED_DIM = 64
BIO_DIM = 128
B, L = 4096, 50
TOTAL = B * L

NC = 2
NS = 16
NW = NC * NS
PER_W = TOTAL // NW
CHUNK = 128
ITERS = PER_W // CHUNK

T_BLK = 32768
T_GRID = -(-NUM_MOTIFS // T_BLK)


def _transpose_body(tt_ref, out_ref):
    xt = tt_ref[...].T
    out_ref[...] = jnp.concatenate(
        [xt, jnp.zeros((T_BLK, 2 * EMBED_DIM - EMBED_DIM), jnp.float32)], axis=1
    )


def _tc_transpose(table_t):
    return pl.pallas_call(
        _transpose_body,
        grid=(T_GRID,),
        in_specs=[pl.BlockSpec((EMBED_DIM, T_BLK), lambda i: (0, i))],
        out_specs=pl.BlockSpec((T_BLK, 2 * EMBED_DIM), lambda i: (i, 0)),
        out_shape=jax.ShapeDtypeStruct((NUM_MOTIFS, 2 * EMBED_DIM), jnp.float32),
    )(table_t)


def _sc_gather_body(table_hbm, ids_hbm, out_hbm, idx_v, buf0, buf1,
                    gsem0, gsem1, ssem0, ssem1):
    wid = lax.axis_index("s") * NC + lax.axis_index("c")
    base = wid * PER_W
    pltpu.sync_copy(ids_hbm.at[wid], idx_v)

    bufs = (buf0, buf1)
    gsems = (gsem0, gsem1)
    ssems = (ssem0, ssem1)

    def store(g):
        return pltpu.async_copy(
            bufs[g % 2].at[:, pl.ds(0, EMBED_DIM)],
            out_hbm.at[pl.ds(base + g * CHUNK, CHUNK), pl.ds(0, EMBED_DIM)],
            ssems[g % 2],
        )

    def gather(g):
        return pltpu.async_copy(table_hbm.at[idx_v.at[g]], bufs[g % 2],
                                gsems[g % 2])

    hg = [None] * ITERS
    hs = [None] * ITERS
    for g in range(ITERS):
        if g >= 2:
            hs[g - 2].wait()
        hg[g] = gather(g)
        if g >= 1:
            hg[g - 1].wait()
            hs[g - 1] = store(g - 1)
    hg[ITERS - 1].wait()
    hs[ITERS - 1] = store(ITERS - 1)
    hs[ITERS - 2].wait()
    hs[ITERS - 1].wait()


def _sc_gather(table128, ids3d):
    mesh = plsc.VectorSubcoreMesh(
        core_axis_name="c", subcore_axis_name="s", num_cores=NC, num_subcores=NS
    )
    return pl.kernel(
        _sc_gather_body,
        out_type=jax.ShapeDtypeStruct((TOTAL, 2 * EMBED_DIM), jnp.float32),
        mesh=mesh,
        scratch_types=[
            pltpu.VMEM((ITERS, CHUNK), jnp.int32),
            pltpu.VMEM((CHUNK, 2 * EMBED_DIM), jnp.float32),
            pltpu.VMEM((CHUNK, 2 * EMBED_DIM), jnp.float32),
            pltpu.SemaphoreType.DMA,
            pltpu.SemaphoreType.DMA,
            pltpu.SemaphoreType.DMA,
            pltpu.SemaphoreType.DMA,
        ],
        compiler_params=pltpu.CompilerParams(use_tc_tiling_on_sc=False),
    )(table128, ids3d)


def _tc_body(bio_ref, wt_ref, b_ref, g_ref, out_ref):
    acc = jnp.dot(bio_ref[...], wt_ref[...], preferred_element_type=jnp.float32)
    acc = acc + b_ref[...] + g_ref[:, :EMBED_DIM]
    out_ref[...] = acc.T[None]


def _tc_proj_add(bio2d, wt, b2d, gathered):
    return pl.pallas_call(
        _tc_body,
        grid=(L,),
        in_specs=[
            pl.BlockSpec((B, BIO_DIM), lambda i: (i, 0)),
            pl.BlockSpec((BIO_DIM, EMBED_DIM), lambda i: (0, 0)),
            pl.BlockSpec((1, EMBED_DIM), lambda i: (0, 0)),
            pl.BlockSpec((B, 2 * EMBED_DIM), lambda i: (i, 0)),
        ],
        out_specs=pl.BlockSpec((1, EMBED_DIM, B), lambda i: (i, 0, 0)),
        out_shape=jax.ShapeDtypeStruct((L, EMBED_DIM, B), jnp.float32),
    )(bio2d, wt, b2d, gathered)


@jax.jit
def kernel(motif_ids, biological_features, emb_table, W, b):
    table128 = _tc_transpose(emb_table.T)
    ids3d = motif_ids.T.reshape(NW, ITERS, CHUNK).astype(jnp.int32)
    gathered = _sc_gather(table128, ids3d)
    bio2d = biological_features.transpose(1, 0, 2).reshape(TOTAL, BIO_DIM)
    out_t = _tc_proj_add(bio2d, W.T, b.reshape(1, EMBED_DIM), gathered)
    return out_t.transpose(2, 0, 1)


# speedup vs baseline: 1.3519x; 1.0517x over previous
import jax
import jax.numpy as jnp
from jax import lax
from jax.experimental import pallas as pl
from jax.experimental.pallas import tpu as pltpu
from jax.experimental.pallas import tpu_sc as plsc

NUM_MOTIFS = 1000000
EMBED_DIM = 64
BIO_DIM = 128
B, L = 4096, 50
TOTAL = B * L

NC = 2
NS = 16
NW = NC * NS
PER_W = TOTAL // NW
CHUNK = 128
ITERS = PER_W // CHUNK

T_BLK = 32768
T_GRID = -(-NUM_MOTIFS // T_BLK)


def _transpose_body(tt_ref, out_ref):
    xt = tt_ref[...].T
    out_ref[...] = jnp.concatenate(
        [xt, jnp.zeros((T_BLK, 2 * EMBED_DIM - EMBED_DIM), jnp.float32)], axis=1
    )


def _tc_transpose(table_t):
    return pl.pallas_call(
        _transpose_body,
        grid=(T_GRID,),
        in_specs=[pl.BlockSpec((EMBED_DIM, T_BLK), lambda i: (0, i))],
        out_specs=pl.BlockSpec((T_BLK, 2 * EMBED_DIM), lambda i: (i, 0)),
        out_shape=jax.ShapeDtypeStruct((NUM_MOTIFS, 2 * EMBED_DIM), jnp.float32),
    )(table_t)


def _sc_gather_body(table_hbm, ids_hbm, out_hbm, idx_v, buf0, buf1,
                    gsem0, gsem1, ssem0, ssem1):
    wid = lax.axis_index("s") * NC + lax.axis_index("c")
    base = wid * PER_W
    pltpu.sync_copy(ids_hbm.at[wid], idx_v)

    bufs = (buf0, buf1)
    gsems = (gsem0, gsem1)
    ssems = (ssem0, ssem1)

    def store(g):
        return pltpu.async_copy(
            bufs[g % 2],
            out_hbm.at[pl.ds(base + g * CHUNK, CHUNK), pl.ds(0, EMBED_DIM)],
            ssems[g % 2],
        )

    def gather(g):
        return pltpu.async_copy(table_hbm.at[idx_v.at[g]], bufs[g % 2],
                                gsems[g % 2])

    hg = [None] * ITERS
    hs = [None] * ITERS
    for g in range(ITERS):
        if g >= 2:
            hs[g - 2].wait()
        hg[g] = gather(g)
        if g >= 1:
            hg[g - 1].wait()
            hs[g - 1] = store(g - 1)
    hg[ITERS - 1].wait()
    hs[ITERS - 1] = store(ITERS - 1)
    hs[ITERS - 2].wait()
    hs[ITERS - 1].wait()


def _sc_gather(table128, ids3d):
    mesh = plsc.VectorSubcoreMesh(
        core_axis_name="c", subcore_axis_name="s", num_cores=NC, num_subcores=NS
    )
    return pl.kernel(
        _sc_gather_body,
        out_type=jax.ShapeDtypeStruct((TOTAL, 2 * EMBED_DIM), jnp.float32),
        mesh=mesh,
        scratch_types=[
            pltpu.VMEM((ITERS, CHUNK), jnp.int32),
            pltpu.VMEM((CHUNK, EMBED_DIM), jnp.float32),
            pltpu.VMEM((CHUNK, EMBED_DIM), jnp.float32),
            pltpu.SemaphoreType.DMA,
            pltpu.SemaphoreType.DMA,
            pltpu.SemaphoreType.DMA,
            pltpu.SemaphoreType.DMA,
        ],
        compiler_params=pltpu.CompilerParams(use_tc_tiling_on_sc=False),
    )(table128, ids3d)


def _tc_body(bio_ref, wt_ref, b_ref, g_ref, out_ref):
    acc = jnp.dot(bio_ref[...], wt_ref[...], preferred_element_type=jnp.float32)
    acc = acc + b_ref[...] + g_ref[:, :EMBED_DIM]
    out_ref[...] = acc.T[None]


def _tc_proj_add(bio2d, wt, b2d, gathered):
    return pl.pallas_call(
        _tc_body,
        grid=(L,),
        in_specs=[
            pl.BlockSpec((B, BIO_DIM), lambda i: (i, 0)),
            pl.BlockSpec((BIO_DIM, EMBED_DIM), lambda i: (0, 0)),
            pl.BlockSpec((1, EMBED_DIM), lambda i: (0, 0)),
            pl.BlockSpec((B, 2 * EMBED_DIM), lambda i: (i, 0)),
        ],
        out_specs=pl.BlockSpec((1, EMBED_DIM, B), lambda i: (i, 0, 0)),
        out_shape=jax.ShapeDtypeStruct((L, EMBED_DIM, B), jnp.float32),
    )(bio2d, wt, b2d, gathered)


@jax.jit
def kernel(motif_ids, biological_features, emb_table, W, b):
    table128 = _tc_transpose(emb_table.T)
    table64 = table128.reshape(2 * NUM_MOTIFS, EMBED_DIM)
    ids3d = (motif_ids.T.astype(jnp.int32) << 1).reshape(NW, ITERS, CHUNK)
    gathered = _sc_gather(table64, ids3d)
    bio2d = biological_features.transpose(1, 0, 2).reshape(TOTAL, BIO_DIM)
    out_t = _tc_proj_add(bio2d, W.T, b.reshape(1, EMBED_DIM), gathered)
    return out_t.transpose(2, 0, 1)
